# baseline (speedup 1.0000x reference)
import functools

import jax
import jax.numpy as jnp
from jax import lax
from jax.experimental import pallas as pl
from jax.experimental.pallas import tpu as pltpu
from jax.experimental.pallas import tpu_sc as plsc

NC, NS = 2, 16
NW = NC * NS


def _sc_embed(x3, tok, pos3):
    B, nh, HL = x3.shape
    E = tok.shape[1]
    SPW = B // NW
    NITEMS = nh * SPW
    NBUF = 4

    mesh = plsc.VectorSubcoreMesh(core_axis_name="c", subcore_axis_name="s")

    @functools.partial(
        pl.kernel,
        out_type=jax.ShapeDtypeStruct((B, nh, HL, E), jnp.float32),
        mesh=mesh,
        compiler_params=pltpu.CompilerParams(use_tc_tiling_on_sc=False),
        scratch_types=[
            pltpu.VMEM((SPW, nh, HL), jnp.int32),
            pltpu.VMEM((NBUF, HL, E), jnp.float32),
            pltpu.VMEM((nh, HL, E), jnp.float32),
            pltpu.SemaphoreType.DMA((NBUF,)),
            pltpu.SemaphoreType.DMA((NBUF,)),
        ],
    )
    def k(x3_hbm, tok_hbm, pos3_hbm, out_hbm, idx_all, rows_v, pos_v, gsem, osem):
        wid = lax.axis_index("s") * NC + lax.axis_index("c")
        base_seq = wid * SPW

        pltpu.sync_copy(pos3_hbm, pos_v)
        pltpu.sync_copy(x3_hbm.at[pl.ds(base_seq, SPW)], idx_all)

        def start_gather(t, jb, h):
            s = t // 2
            pltpu.async_copy(
                tok_hbm.at[idx_all.at[s, h]], rows_v.at[jb], gsem.at[jb])

        def wait_gather(t, jb, h):
            s = t // 2
            pltpu.make_async_copy(
                tok_hbm.at[idx_all.at[s, h]], rows_v.at[jb], gsem.at[jb]).wait()

        def add_pos(jb, h):
            @pl.loop(0, HL)
            def _(r):
                for c in range(E // 16):
                    sl = pl.ds(c * 16, 16)
                    plsc.addupdate(rows_v.at[jb, r, sl], pos_v[h, r, sl])

        def start_out(t, jb, h):
            s = t // 2
            pltpu.async_copy(
                rows_v.at[jb], out_hbm.at[base_seq + s, h], osem.at[jb])

        def wait_out(t, jb, h):
            s = t // 2
            pltpu.make_async_copy(
                rows_v.at[jb], out_hbm.at[base_seq + s, h], osem.at[jb]).wait()

        def consume_and_prefetch(t, jb, h, outwait, start):
            wait_gather(t, jb, h)
            add_pos(jb, h)
            start_out(t, jb, h)
            fut = t + 3
            fb = (jb + 3) % NBUF
            fh = (h + 1) % 2
            if outwait:
                wait_out(fut - NBUF, fb, fh)
            if start:
                start_gather(fut, fb, fh)

        for t in range(3):
            start_gather(t, t % NBUF, t % 2)

        consume_and_prefetch(0, 0, 0, False, True)
        consume_and_prefetch(1, 1, 1, True, True)
        consume_and_prefetch(2, 2, 0, True, True)
        consume_and_prefetch(3, 3, 1, True, True)

        @pl.loop(4, NITEMS - 4, step=4)
        def _(t0):
            for j in range(4):
                consume_and_prefetch(t0 + j, j, j % 2, True, True)

        t0 = NITEMS - 4
        consume_and_prefetch(t0 + 0, 0, 0, True, True)
        consume_and_prefetch(t0 + 1, 1, 1, False, False)
        consume_and_prefetch(t0 + 2, 2, 0, False, False)
        consume_and_prefetch(t0 + 3, 3, 1, False, False)

        for t in range(NITEMS - 4, NITEMS):
            wait_out(t, t % NBUF, t % 2)

    return k(x3, tok, pos3)


@jax.jit
def kernel(x, token_table, pos_table):
    B, L = x.shape
    E = token_table.shape[1]
    HL = L // 2
    x3 = x.astype(jnp.int32).reshape(B, 2, HL)
    pos3 = pos_table[:L].astype(jnp.float32).reshape(2, HL, E)
    out4 = _sc_embed(x3, token_table, pos3)
    return out4.reshape(B, L, E)
